# baseline (speedup 1.0000x reference)
import jax
import jax.numpy as jnp
from jax.experimental import pallas as pl

_BS = 256


def _add_kernel(x_ref, w_ref, o_ref):
    o_ref[...] = x_ref[...] + w_ref[...][None, :, :]


def kernel(x, weight):
    batch, seq_len, dim = x.shape
    w = jax.lax.slice(weight, (0, 0), (seq_len, dim))
    grid = (seq_len // _BS,)
    return pl.pallas_call(
        _add_kernel,
        grid=grid,
        in_specs=[
            pl.BlockSpec((batch, _BS, dim), lambda i: (0, i, 0)),
            pl.BlockSpec((_BS, dim), lambda i: (i, 0)),
        ],
        out_specs=pl.BlockSpec((batch, _BS, dim), lambda i: (0, i, 0)),
        out_shape=jax.ShapeDtypeStruct((batch, seq_len, dim), x.dtype),
    )(x, w)


# speedup vs baseline: 1.0074x; 1.0074x over previous
import jax
import jax.numpy as jnp
from jax.experimental import pallas as pl

_BS = 512


def _add_kernel(x_ref, w_ref, o_ref):
    o_ref[...] = x_ref[...] + w_ref[...][None, :, :]


def kernel(x, weight):
    batch, seq_len, dim = x.shape
    w = jax.lax.slice(weight, (0, 0), (seq_len, dim))
    grid = (seq_len // _BS,)
    return pl.pallas_call(
        _add_kernel,
        grid=grid,
        in_specs=[
            pl.BlockSpec((batch, _BS, dim), lambda i: (0, i, 0)),
            pl.BlockSpec((_BS, dim), lambda i: (i, 0)),
        ],
        out_specs=pl.BlockSpec((batch, _BS, dim), lambda i: (0, i, 0)),
        out_shape=jax.ShapeDtypeStruct((batch, seq_len, dim), x.dtype),
    )(x, w)
